# baseline (speedup 1.0000x reference)
import functools

import jax
import jax.numpy as jnp
from jax import lax
from jax.experimental import pallas as pl
from jax.experimental.pallas import tpu as pltpu
from jax.experimental.pallas import tpu_sc as plsc

D_MODEL = 64
LN_EPS = 1e-5
SEQ = 128
BATCH = 4096
NROWS = SEQ * BATCH

NC = 2
NS = 16
L = 16
NW = NC * NS

ROWS_PER_W = NROWS // NW
CHUNK = 128
NCHUNKS = ROWS_PER_W // CHUNK
NSEG = D_MODEL // L

def _lane_sum(v):
    lanes = lax.iota(jnp.int32, L)
    dnums = lax.GatherDimensionNumbers(
        offset_dims=(), collapsed_slice_dims=(0,), start_index_map=(0,)
    )
    for sh in (8, 4, 2, 1):
        perm = lax.gather(
            v, (lanes ^ sh)[:, None], dnums, (1,),
            mode=lax.GatherScatterMode.PROMISE_IN_BOUNDS,
        )
        v = v + perm
    return v


_mesh = plsc.VectorSubcoreMesh(
    core_axis_name="c", subcore_axis_name="s", num_cores=NC, num_subcores=NS
)


@functools.partial(
    pl.kernel,
    out_type=jax.ShapeDtypeStruct((NROWS, D_MODEL), jnp.float32),
    mesh=_mesh,
    scratch_types=[
        pltpu.VMEM((CHUNK,), jnp.int32),
        pltpu.VMEM((CHUNK, D_MODEL), jnp.float32),
        pltpu.VMEM((D_MODEL,), jnp.float32),
        pltpu.VMEM((D_MODEL,), jnp.float32),
        pltpu.VMEM((D_MODEL,), jnp.float32),
        pltpu.SemaphoreType.DMA,
    ],
    compiler_params=pltpu.CompilerParams(use_tc_tiling_on_sc=False),
)
def _sc_embed_ln(x_hbm, tab_hbm, pos_hbm, gamma_hbm, beta_hbm, out_hbm,
                 idx_v, rows_v, pos_v, gamma_v, beta_v, sem):
    wid = lax.axis_index("s") * NC + lax.axis_index("c")
    base = wid * ROWS_PER_W

    pltpu.sync_copy(gamma_hbm, gamma_v)
    pltpu.sync_copy(beta_hbm, beta_v)
    gseg = [gamma_v[pl.ds(k * L, L)] for k in range(NSEG)]
    bseg = [beta_v[pl.ds(k * L, L)] for k in range(NSEG)]

    def chunk_body(c, carry):
        row0 = base + c * CHUNK
        pltpu.sync_copy(x_hbm.at[pl.ds(row0, CHUNK)], idx_v)
        pltpu.async_copy(tab_hbm.at[idx_v], rows_v, sem).wait()
        s_pos = lax.shift_right_logical(row0, 12)
        pltpu.sync_copy(pos_hbm.at[s_pos], pos_v)
        pseg = [pos_v[pl.ds(k * L, L)] for k in range(NSEG)]

        def row_body(r, carry2):
            e = [rows_v[r, pl.ds(k * L, L)] * 8.0 + pseg[k]
                 for k in range(NSEG)]
            ssum = (e[0] + e[1]) + (e[2] + e[3])
            ssq = ((e[0] * e[0] + e[1] * e[1])
                   + (e[2] * e[2] + e[3] * e[3]))
            mean_v = _lane_sum(ssum) * (1.0 / D_MODEL)
            var_v = _lane_sum(ssq) * (1.0 / D_MODEL) - mean_v * mean_v
            a_v = var_v + LN_EPS
            yi = jnp.full((L,), 0x5F3759DF, jnp.int32) - lax.shift_right_logical(
                lax.bitcast_convert_type(a_v, jnp.int32), 1
            )
            y = lax.bitcast_convert_type(yi, jnp.float32)
            h_v = a_v * -0.5
            y = y * (y * y * h_v + 1.5)
            y = y * (y * y * h_v + 1.5)
            rstd_v = y
            for k in range(NSEG):
                rows_v[r, pl.ds(k * L, L)] = (
                    (e[k] - mean_v) * rstd_v * gseg[k] + bseg[k]
                )
            return carry2

        lax.fori_loop(0, CHUNK, row_body, 0, unroll=2)
        pltpu.sync_copy(rows_v, out_hbm.at[pl.ds(row0, CHUNK)])
        return carry

    lax.fori_loop(0, NCHUNKS, chunk_body, 0)


def kernel(x, token_table, pos_table, ln_gamma, ln_beta):
    x_flat = x.reshape(NROWS).astype(jnp.int32)
    out = _sc_embed_ln(x_flat, token_table, pos_table, ln_gamma, ln_beta)
    return out.reshape(SEQ, BATCH, D_MODEL)
